# Initial kernel scaffold; baseline (speedup 1.0000x reference)
import jax
import jax.numpy as jnp
from jax.experimental import pallas as pl


def kernel(nodes, neighbors, W, a_src, a_tgt, b):
    raise NotImplementedError("write your pallas kernel here")


# speedup vs baseline: 1.3326x; 1.3326x over previous
import jax
import jax.numpy as jnp
from jax.experimental import pallas as pl

N = 10000
K = 32
F = 128
BLOCK_N = 400


def _gat_kernel(nodes_ref, nb_ref, w_ref, asrc_ref, atgt_ref, b_ref, out_ref):
    W = w_ref[...]
    v_src = asrc_ref[...] @ W
    v_tgt = atgt_ref[...] @ W

    x = nodes_ref[...]
    nb = nb_ref[...]

    s_tgt = (x * v_tgt).sum(axis=-1)
    s_src = (nb * v_src[None, :, :]).sum(axis=-1)

    e = s_src + s_tgt[:, None]
    e = jnp.where(e > 0, e, 0.2 * e)
    ex = jnp.exp(e)
    attn = ex / (ex.sum(axis=-1, keepdims=True) + 1e-16)

    agg = (nb * attn[:, :, None]).sum(axis=1)

    out = jax.lax.dot_general(agg, W, (((1,), (1,)), ((), ())),
                              preferred_element_type=jnp.float32)
    out = out + b_ref[...]
    out_ref[...] = jnp.where(out > 0, out, jnp.exp(out) - 1.0)


def kernel(nodes, neighbors, W, a_src, a_tgt, b):
    a_src2 = a_src.reshape(1, F)
    a_tgt2 = a_tgt.reshape(1, F)
    b2 = b.reshape(1, F)
    grid = (N // BLOCK_N,)
    return pl.pallas_call(
        _gat_kernel,
        grid=grid,
        in_specs=[
            pl.BlockSpec((BLOCK_N, F), lambda i: (i, 0)),
            pl.BlockSpec((BLOCK_N, K, F), lambda i: (i, 0, 0)),
            pl.BlockSpec((F, F), lambda i: (0, 0)),
            pl.BlockSpec((1, F), lambda i: (0, 0)),
            pl.BlockSpec((1, F), lambda i: (0, 0)),
            pl.BlockSpec((1, F), lambda i: (0, 0)),
        ],
        out_specs=pl.BlockSpec((BLOCK_N, F), lambda i: (i, 0)),
        out_shape=jax.ShapeDtypeStruct((N, F), jnp.float32),
    )(nodes, neighbors, W, a_src2, a_tgt2, b2)


# speedup vs baseline: 1.5739x; 1.1811x over previous
import jax
import jax.numpy as jnp
from jax.experimental import pallas as pl
import jax.experimental.pallas.tpu as pltpu

N = 10000
K = 32
F = 128
BLOCK_N = 400


def _fold_vecs_kernel(w_ref, asrc_ref, atgt_ref, vsrc_ref, vtgt_ref):
    W = w_ref[...]
    vsrc_ref[...] = asrc_ref[...] @ W
    vtgt_ref[...] = atgt_ref[...] @ W


def _gat_kernel(nodes_ref, nb_ref, w_ref, vsrc_ref, vtgt_ref, b_ref, out_ref,
                s_scr, st_scr):
    p = pl.program_id(1)

    @pl.when(p == 0)
    def _scores():
        nb = nb_ref[...]
        x = nodes_ref[...]
        s_scr[...] = (nb * vsrc_ref[...][None, :, :]).sum(axis=-1)
        st_scr[...] = (x * vtgt_ref[...]).sum(axis=-1, keepdims=True)

    @pl.when(p == 1)
    def _aggregate():
        nb = nb_ref[...]
        e = s_scr[...] + st_scr[...]
        e = jnp.where(e > 0, e, 0.2 * e)
        ex = jnp.exp(e)
        denom = ex.sum(axis=-1, keepdims=True) + 1e-16
        agg = (nb * ex[:, :, None]).sum(axis=1) / denom
        out = jax.lax.dot_general(agg, w_ref[...], (((1,), (1,)), ((), ())),
                                  preferred_element_type=jnp.float32)
        out = out + b_ref[...]
        out_ref[...] = jnp.where(out > 0, out, jnp.exp(out) - 1.0)


def kernel(nodes, neighbors, W, a_src, a_tgt, b):
    a_src2 = a_src.reshape(1, F)
    a_tgt2 = a_tgt.reshape(1, F)
    b2 = b.reshape(1, F)

    v_src, v_tgt = pl.pallas_call(
        _fold_vecs_kernel,
        out_shape=[
            jax.ShapeDtypeStruct((1, F), jnp.float32),
            jax.ShapeDtypeStruct((1, F), jnp.float32),
        ],
    )(W, a_src2, a_tgt2)

    grid = (N // BLOCK_N, 2)
    return pl.pallas_call(
        _gat_kernel,
        grid=grid,
        in_specs=[
            pl.BlockSpec((BLOCK_N, F), lambda i, p: (i, 0)),
            pl.BlockSpec((BLOCK_N, K, F), lambda i, p: (i, 0, 0)),
            pl.BlockSpec((F, F), lambda i, p: (0, 0)),
            pl.BlockSpec((1, F), lambda i, p: (0, 0)),
            pl.BlockSpec((1, F), lambda i, p: (0, 0)),
            pl.BlockSpec((1, F), lambda i, p: (0, 0)),
        ],
        out_specs=pl.BlockSpec((BLOCK_N, F), lambda i, p: (i, 0)),
        out_shape=jax.ShapeDtypeStruct((N, F), jnp.float32),
        scratch_shapes=[
            pltpu.VMEM((BLOCK_N, K), jnp.float32),
            pltpu.VMEM((BLOCK_N, 1), jnp.float32),
        ],
    )(nodes, neighbors, W, v_src, v_tgt, b2)


# speedup vs baseline: 1.7625x; 1.1198x over previous
import jax
import jax.numpy as jnp
from jax.experimental import pallas as pl
import jax.experimental.pallas.tpu as pltpu

N = 10000
K = 32
F = 128
BLOCK_N = 1000


def _fold_vecs_kernel(w_ref, asrc_ref, atgt_ref, vsrc_ref, vtgt_ref):
    W = w_ref[...]
    vsrc_ref[...] = asrc_ref[...] @ W
    vtgt_ref[...] = atgt_ref[...] @ W


def _gat_kernel(nodes_ref, nb_ref, w_ref, vsrc_ref, vtgt_ref, b_ref, out_ref,
                s_scr, st_scr):
    p = pl.program_id(1)

    @pl.when(p == 0)
    def _scores():
        nb = nb_ref[...]
        x = nodes_ref[...]
        s_scr[...] = (nb * vsrc_ref[...][None, :, :]).sum(axis=-1)
        st_scr[...] = (x * vtgt_ref[...]).sum(axis=-1, keepdims=True)

    @pl.when(p == 1)
    def _aggregate():
        nb = nb_ref[...]
        e = s_scr[...] + st_scr[...]
        e = jnp.where(e > 0, e, 0.2 * e)
        ex = jnp.exp(e)
        denom = ex.sum(axis=-1, keepdims=True) + 1e-16
        agg = (nb * ex[:, :, None]).sum(axis=1) / denom
        out = jax.lax.dot_general(agg, w_ref[...], (((1,), (1,)), ((), ())),
                                  preferred_element_type=jnp.float32)
        out = out + b_ref[...]
        out_ref[...] = jnp.where(out > 0, out, jnp.exp(out) - 1.0)


def kernel(nodes, neighbors, W, a_src, a_tgt, b):
    a_src2 = a_src.reshape(1, F)
    a_tgt2 = a_tgt.reshape(1, F)
    b2 = b.reshape(1, F)

    v_src, v_tgt = pl.pallas_call(
        _fold_vecs_kernel,
        out_shape=[
            jax.ShapeDtypeStruct((1, F), jnp.float32),
            jax.ShapeDtypeStruct((1, F), jnp.float32),
        ],
    )(W, a_src2, a_tgt2)

    grid = (N // BLOCK_N, 2)
    return pl.pallas_call(
        _gat_kernel,
        grid=grid,
        in_specs=[
            pl.BlockSpec((BLOCK_N, F), lambda i, p: (i, 0)),
            pl.BlockSpec((BLOCK_N, K, F), lambda i, p: (i, 0, 0)),
            pl.BlockSpec((F, F), lambda i, p: (0, 0)),
            pl.BlockSpec((1, F), lambda i, p: (0, 0)),
            pl.BlockSpec((1, F), lambda i, p: (0, 0)),
            pl.BlockSpec((1, F), lambda i, p: (0, 0)),
        ],
        out_specs=pl.BlockSpec((BLOCK_N, F), lambda i, p: (i, 0)),
        out_shape=jax.ShapeDtypeStruct((N, F), jnp.float32),
        scratch_shapes=[
            pltpu.VMEM((BLOCK_N, K), jnp.float32),
            pltpu.VMEM((BLOCK_N, 1), jnp.float32),
        ],
    )(nodes, neighbors, W, v_src, v_tgt, b2)


# speedup vs baseline: 1.9603x; 1.1123x over previous
import jax
import jax.numpy as jnp
from jax.experimental import pallas as pl
import jax.experimental.pallas.tpu as pltpu

N = 10000
K = 32
F = 128
BLOCK_N = 1000


def _fold_vecs_kernel(w_ref, asrc_ref, atgt_ref, vsrc_ref, vtgt_ref):
    W = w_ref[...]
    vsrc_ref[...] = asrc_ref[...] @ W
    vtgt_ref[...] = atgt_ref[...] @ W


def _gat_kernel(nodes_ref, nb_ref, w_ref, vsrc_ref, vtgt_ref, b_ref, out_ref,
                s_scr, st_scr):
    nb = nb_ref[...]
    x = nodes_ref[...]
    s_scr[...] = (nb * vsrc_ref[...][None, :, :]).sum(axis=-1)
    st_scr[...] = (x * vtgt_ref[...]).sum(axis=-1, keepdims=True)

    zero = pl.program_id(0) * 0
    s = s_scr[pl.ds(zero, BLOCK_N), :]
    st = st_scr[pl.ds(zero, BLOCK_N), :]

    e = s + st
    e = jnp.where(e > 0, e, 0.2 * e)
    ex = jnp.exp(e)
    denom = ex.sum(axis=-1, keepdims=True) + 1e-16
    agg = (nb * ex[:, :, None]).sum(axis=1) / denom
    out = jax.lax.dot_general(agg, w_ref[...], (((1,), (1,)), ((), ())),
                              preferred_element_type=jnp.float32)
    out = out + b_ref[...]
    out_ref[...] = jnp.where(out > 0, out, jnp.exp(out) - 1.0)


def kernel(nodes, neighbors, W, a_src, a_tgt, b):
    a_src2 = a_src.reshape(1, F)
    a_tgt2 = a_tgt.reshape(1, F)
    b2 = b.reshape(1, F)

    v_src, v_tgt = pl.pallas_call(
        _fold_vecs_kernel,
        out_shape=[
            jax.ShapeDtypeStruct((1, F), jnp.float32),
            jax.ShapeDtypeStruct((1, F), jnp.float32),
        ],
    )(W, a_src2, a_tgt2)

    grid = (N // BLOCK_N,)
    return pl.pallas_call(
        _gat_kernel,
        grid=grid,
        in_specs=[
            pl.BlockSpec((BLOCK_N, F), lambda i: (i, 0)),
            pl.BlockSpec((BLOCK_N, K, F), lambda i: (i, 0, 0)),
            pl.BlockSpec((F, F), lambda i: (0, 0)),
            pl.BlockSpec((1, F), lambda i: (0, 0)),
            pl.BlockSpec((1, F), lambda i: (0, 0)),
            pl.BlockSpec((1, F), lambda i: (0, 0)),
        ],
        out_specs=pl.BlockSpec((BLOCK_N, F), lambda i: (i, 0)),
        out_shape=jax.ShapeDtypeStruct((N, F), jnp.float32),
        scratch_shapes=[
            pltpu.VMEM((BLOCK_N, K), jnp.float32),
            pltpu.VMEM((BLOCK_N, 1), jnp.float32),
        ],
    )(nodes, neighbors, W, v_src, v_tgt, b2)
